# baseline (speedup 1.0000x reference)
import functools

import jax
import jax.numpy as jnp
from jax import lax
from jax.experimental import pallas as pl
from jax.experimental.pallas import tpu as pltpu
from jax.experimental.pallas import tpu_sc as plsc

N = 10000
D = 128
H = 128
E = 320000
EPS = 1e-5

NC = 2
NS = 16
NW = NC * NS
RPT = 632
NPAD = NS * RPT
CHUNK = 128
NCHUNK = 80
HALF = 40
EPAD = NW * NCHUNK * CHUNK

_SELU_SCALE = 1.0507009873554805
_SELU_ALPHA = 1.6732632423543772

_HI = lax.Precision.HIGHEST


def _selu(x):
    neg = _SELU_ALPHA * (jnp.exp(jnp.minimum(x, 0.0)) - 1.0)
    return _SELU_SCALE * jnp.where(x > 0, x, neg)


def _batchnorm(s, g, beta):
    m = jnp.mean(s, axis=0, keepdims=True)
    v = jnp.mean((s - m) ** 2, axis=0, keepdims=True)
    return (s - m) * lax.rsqrt(v + EPS) * g + beta


def _pre_body(x_ref, w00_ref, w10a_ref, q0_ref, r1_ref):
    x = x_ref[...]
    q0_ref[...] = jnp.dot(x, w00_ref[...], precision=_HI,
                          preferred_element_type=jnp.float32)
    r1_ref[...] = jnp.dot(x, w10a_ref[...], precision=_HI,
                          preferred_element_type=jnp.float32)


def _mid_body(p0_ref, p1_ref, ba_ref, wb_ref, bb_ref, g_ref, beta_ref,
              r1_ref, w10b_ref, c0_ref, q1_ref):
    agg = p0_ref[...] + p1_ref[...]
    hid = jnp.maximum(agg + ba_ref[...], 0.0)
    out = jnp.dot(hid, wb_ref[...], precision=_HI,
                  preferred_element_type=jnp.float32) + bb_ref[...]
    c0 = _batchnorm(_selu(out), g_ref[...], beta_ref[...])
    c0_ref[...] = c0
    q1_ref[...] = r1_ref[...] + jnp.dot(c0, w10b_ref[...], precision=_HI,
                                        preferred_element_type=jnp.float32)


def _post_body(p0_ref, p1_ref, ba_ref, wb_ref, bb_ref, g_ref, beta_ref,
               c1_ref):
    agg = p0_ref[...] + p1_ref[...]
    hid = jnp.maximum(agg + ba_ref[...], 0.0)
    out = jnp.dot(hid, wb_ref[...], precision=_HI,
                  preferred_element_type=jnp.float32) + bb_ref[...]
    c1_ref[...] = _batchnorm(_selu(out), g_ref[...], beta_ref[...])


_f32 = lambda *s: jax.ShapeDtypeStruct(s, jnp.float32)

_pre = pl.pallas_call(
    _pre_body, out_shape=[_f32(N, H), _f32(N, H)])

_mid = pl.pallas_call(
    _mid_body, out_shape=[_f32(N, H), _f32(N, H)])

_post = pl.pallas_call(
    _post_body, out_shape=_f32(N, H))


def _spmm_body(q_hbm, src_hbm, dst_hbm, zeros_hbm, out0_hbm, out1_hbm,
               idx_s, idx_d, rows_a, rows_b, acc, sem_a, sem_b):
    cid = lax.axis_index("c")
    sid = lax.axis_index("s")
    wid = cid * NS + sid
    r0 = sid * RPT
    pltpu.sync_copy(zeros_hbm.at[pl.ds(r0, RPT)], acc.at[pl.ds(r0, RPT)])
    plsc.subcore_barrier()

    def body(i, carry):
        pltpu.sync_copy(src_hbm.at[wid, i], idx_s)
        pltpu.sync_copy(dst_hbm.at[wid, i], idx_d)
        pltpu.async_copy(q_hbm.at[idx_s], rows_a, sem_a).wait()
        pltpu.sync_copy(rows_a, acc.at[idx_d], add=True)
        return carry

    lax.fori_loop(0, NCHUNK, body, 0)
    plsc.subcore_barrier()

    @pl.when(cid == 0)
    def _():
        pltpu.sync_copy(acc.at[pl.ds(r0, RPT)], out0_hbm.at[pl.ds(r0, RPT)])

    @pl.when(cid == 1)
    def _():
        pltpu.sync_copy(acc.at[pl.ds(r0, RPT)], out1_hbm.at[pl.ds(r0, RPT)])


_spmm = pl.kernel(
    _spmm_body,
    out_type=[_f32(NPAD, H), _f32(NPAD, H)],
    mesh=plsc.VectorSubcoreMesh(core_axis_name="c", subcore_axis_name="s"),
    scratch_types=[
        pltpu.VMEM((CHUNK,), jnp.int32),
        pltpu.VMEM((CHUNK,), jnp.int32),
        pltpu.VMEM((CHUNK, H), jnp.float32),
        pltpu.VMEM((CHUNK, H), jnp.float32),
        pltpu.VMEM_SHARED((NPAD, H), jnp.float32),
        pltpu.SemaphoreType.DMA,
        pltpu.SemaphoreType.DMA,
    ],
)


def kernel(x, edge_index, W0_0, b0_0, W0_1, b0_1, W1_0, b1_0, W1_1, b1_1,
           g0, beta0, g1, beta1):
    pad = EPAD - E
    src = jnp.concatenate((edge_index[0], jnp.zeros((pad,), jnp.int32)))
    src = src.reshape(NW, NCHUNK, CHUNK)
    dst = jnp.concatenate((edge_index[1], jnp.full((pad,), N, jnp.int32)))
    dst = dst.reshape(NW, NCHUNK, CHUNK)
    zeros = jnp.zeros((NPAD, H), jnp.float32)

    b0_0r = b0_0.reshape(1, H)
    b0_1r = b0_1.reshape(1, H)
    b1_0r = b1_0.reshape(1, H)
    b1_1r = b1_1.reshape(1, H)
    g0r = g0.reshape(1, H)
    beta0r = beta0.reshape(1, H)
    g1r = g1.reshape(1, H)
    beta1r = beta1.reshape(1, H)

    q0, r1 = _pre(x, W0_0, W1_0[:D])
    p0a, p0b = _spmm(q0, src, dst, zeros)
    c0, q1 = _mid(p0a[:N], p0b[:N], b0_0r, W0_1, b0_1r, g0r, beta0r,
                  r1, W1_0[D:])
    p1a, p1b = _spmm(q1, src, dst, zeros)
    c1 = _post(p1a[:N], p1b[:N], b1_0r, W1_1, b1_1r, g1r, beta1r)
    return jnp.concatenate((x, c0, c1), axis=-1)


# speedup vs baseline: 1.1787x; 1.1787x over previous
import functools

import jax
import jax.numpy as jnp
from jax import lax
from jax.experimental import pallas as pl
from jax.experimental.pallas import tpu as pltpu
from jax.experimental.pallas import tpu_sc as plsc

N = 10000
D = 128
H = 128
E = 320000
EPS = 1e-5

NC = 2
NS = 16
NW = NC * NS
RPT = 632
NPAD = NS * RPT
CHUNK = 128
NCHUNK = 80
HALF = 40
EPAD = NW * NCHUNK * CHUNK

_SELU_SCALE = 1.0507009873554805
_SELU_ALPHA = 1.6732632423543772

_HI = lax.Precision.HIGHEST


def _selu(x):
    neg = _SELU_ALPHA * (jnp.exp(jnp.minimum(x, 0.0)) - 1.0)
    return _SELU_SCALE * jnp.where(x > 0, x, neg)


def _batchnorm(s, g, beta):
    m = jnp.mean(s, axis=0, keepdims=True)
    v = jnp.mean((s - m) ** 2, axis=0, keepdims=True)
    return (s - m) * lax.rsqrt(v + EPS) * g + beta


def _pre_body(x_ref, w00_ref, w10a_ref, q0_ref, r1_ref):
    x = x_ref[...]
    q0_ref[...] = jnp.dot(x, w00_ref[...], precision=_HI,
                          preferred_element_type=jnp.float32)
    r1_ref[...] = jnp.dot(x, w10a_ref[...], precision=_HI,
                          preferred_element_type=jnp.float32)


def _mid_body(p0_ref, p1_ref, ba_ref, wb_ref, bb_ref, g_ref, beta_ref,
              r1_ref, w10b_ref, c0_ref, q1_ref):
    agg = p0_ref[...] + p1_ref[...]
    hid = jnp.maximum(agg + ba_ref[...], 0.0)
    out = jnp.dot(hid, wb_ref[...], precision=_HI,
                  preferred_element_type=jnp.float32) + bb_ref[...]
    c0 = _batchnorm(_selu(out), g_ref[...], beta_ref[...])
    c0_ref[...] = c0
    q1_ref[...] = r1_ref[...] + jnp.dot(c0, w10b_ref[...], precision=_HI,
                                        preferred_element_type=jnp.float32)


def _post_body(p0_ref, p1_ref, ba_ref, wb_ref, bb_ref, g_ref, beta_ref,
               c1_ref):
    agg = p0_ref[...] + p1_ref[...]
    hid = jnp.maximum(agg + ba_ref[...], 0.0)
    out = jnp.dot(hid, wb_ref[...], precision=_HI,
                  preferred_element_type=jnp.float32) + bb_ref[...]
    c1_ref[...] = _batchnorm(_selu(out), g_ref[...], beta_ref[...])


_f32 = lambda *s: jax.ShapeDtypeStruct(s, jnp.float32)

_pre = pl.pallas_call(
    _pre_body, out_shape=[_f32(N, H), _f32(N, H)])

_mid = pl.pallas_call(
    _mid_body, out_shape=[_f32(N, H), _f32(N, H)])

_post = pl.pallas_call(
    _post_body, out_shape=_f32(N, H))


def _spmm_body(q_hbm, src_hbm, dst_hbm, zeros_hbm, out0_hbm, out1_hbm,
               idx_s, idx_d, rows_a, rows_b, acc, sem_a, sem_b):
    cid = lax.axis_index("c")
    sid = lax.axis_index("s")
    wid = cid * NS + sid
    r0 = sid * RPT
    pltpu.sync_copy(zeros_hbm.at[pl.ds(r0, RPT)], acc.at[pl.ds(r0, RPT)])
    plsc.subcore_barrier()

    def body(i, carry):
        pltpu.sync_copy(src_hbm.at[wid, i], idx_s)
        pltpu.sync_copy(dst_hbm.at[wid, i], idx_d)
        pltpu.async_copy(q_hbm.at[idx_s], rows_a, sem_a).wait()
        pltpu.sync_copy(rows_a, acc.at[idx_d], add=True)
        return carry

    lax.fori_loop(0, NCHUNK, body, 0)
    plsc.subcore_barrier()

    @pl.when(cid == 0)
    def _():
        pltpu.sync_copy(acc.at[pl.ds(r0, RPT)], out0_hbm.at[pl.ds(r0, RPT)])

    @pl.when(cid == 1)
    def _():
        pltpu.sync_copy(acc.at[pl.ds(r0, RPT)], out1_hbm.at[pl.ds(r0, RPT)])


_spmm = pl.kernel(
    _spmm_body,
    out_type=[_f32(NPAD, H), _f32(NPAD, H)],
    mesh=plsc.VectorSubcoreMesh(core_axis_name="c", subcore_axis_name="s"),
    scratch_types=[
        pltpu.VMEM((CHUNK,), jnp.int32),
        pltpu.VMEM((CHUNK,), jnp.int32),
        pltpu.VMEM((CHUNK, H), jnp.float32),
        pltpu.VMEM((CHUNK, H), jnp.float32),
        pltpu.VMEM_SHARED((NPAD, H), jnp.float32),
        pltpu.SemaphoreType.DMA,
        pltpu.SemaphoreType.DMA,
    ],
)


def kernel(x, edge_index, W0_0, b0_0, W0_1, b0_1, W1_0, b1_0, W1_1, b1_1,
           g0, beta0, g1, beta1):
    epw = E // NW
    ppw = NCHUNK * CHUNK - epw
    src = jnp.concatenate(
        (edge_index[0].reshape(NW, epw), jnp.zeros((NW, ppw), jnp.int32)),
        axis=1).reshape(NW, NCHUNK, CHUNK)
    dst_pad = jnp.broadcast_to(N + (jnp.arange(ppw, dtype=jnp.int32) % (NPAD - N)),
                               (NW, ppw))
    dst = jnp.concatenate(
        (edge_index[1].reshape(NW, epw), dst_pad), axis=1).reshape(NW, NCHUNK, CHUNK)
    zeros = jnp.zeros((NPAD, H), jnp.float32)

    b0_0r = b0_0.reshape(1, H)
    b0_1r = b0_1.reshape(1, H)
    b1_0r = b1_0.reshape(1, H)
    b1_1r = b1_1.reshape(1, H)
    g0r = g0.reshape(1, H)
    beta0r = beta0.reshape(1, H)
    g1r = g1.reshape(1, H)
    beta1r = beta1.reshape(1, H)

    q0, r1 = _pre(x, W0_0, W1_0[:D])
    p0a, p0b = _spmm(q0, src, dst, zeros)
    c0, q1 = _mid(p0a[:N], p0b[:N], b0_0r, W0_1, b0_1r, g0r, beta0r,
                  r1, W1_0[D:])
    p1a, p1b = _spmm(q1, src, dst, zeros)
    c1 = _post(p1a[:N], p1b[:N], b1_0r, W1_1, b1_1r, g1r, beta1r)
    return jnp.concatenate((x, c0, c1), axis=-1)


# speedup vs baseline: 2.8500x; 2.4179x over previous
import functools

import jax
import jax.numpy as jnp
from jax import lax
from jax.experimental import pallas as pl
from jax.experimental.pallas import tpu as pltpu
from jax.experimental.pallas import tpu_sc as plsc

N = 10000
D = 128
H = 128
E = 320000
EPS = 1e-5

NC = 2
NS = 16
NW = NC * NS
RPT = 632
NPAD = NS * RPT
EPW = E // NW
CHUNK = 80
NCHUNK = EPW // CHUNK
EPAD = NW * NCHUNK * CHUNK

_SELU_SCALE = 1.0507009873554805
_SELU_ALPHA = 1.6732632423543772

_HI = lax.Precision.HIGHEST


def _selu(x):
    neg = _SELU_ALPHA * (jnp.exp(jnp.minimum(x, 0.0)) - 1.0)
    return _SELU_SCALE * jnp.where(x > 0, x, neg)


def _batchnorm(s, g, beta):
    m = jnp.mean(s, axis=0, keepdims=True)
    v = jnp.mean((s - m) ** 2, axis=0, keepdims=True)
    return (s - m) * lax.rsqrt(v + EPS) * g + beta


def _pre_body(x_ref, w00_ref, w10a_ref, q0_ref, r1_ref):
    x = x_ref[...]
    q0_ref[...] = jnp.dot(x, w00_ref[...], precision=_HI,
                          preferred_element_type=jnp.float32)
    r1_ref[...] = jnp.dot(x, w10a_ref[...], precision=_HI,
                          preferred_element_type=jnp.float32)


def _mid_body(p0_ref, p1_ref, ba_ref, wb_ref, bb_ref, g_ref, beta_ref,
              r1_ref, w10b_ref, c0_ref, q1_ref):
    agg = p0_ref[...] + p1_ref[...]
    hid = jnp.maximum(agg + ba_ref[...], 0.0)
    out = jnp.dot(hid, wb_ref[...], precision=_HI,
                  preferred_element_type=jnp.float32) + bb_ref[...]
    c0 = _batchnorm(_selu(out), g_ref[...], beta_ref[...])
    c0_ref[...] = c0
    q1_ref[...] = r1_ref[...] + jnp.dot(c0, w10b_ref[...], precision=_HI,
                                        preferred_element_type=jnp.float32)


def _post_body(p0_ref, p1_ref, ba_ref, wb_ref, bb_ref, g_ref, beta_ref,
               c1_ref):
    agg = p0_ref[...] + p1_ref[...]
    hid = jnp.maximum(agg + ba_ref[...], 0.0)
    out = jnp.dot(hid, wb_ref[...], precision=_HI,
                  preferred_element_type=jnp.float32) + bb_ref[...]
    c1_ref[...] = _batchnorm(_selu(out), g_ref[...], beta_ref[...])


_f32 = lambda *s: jax.ShapeDtypeStruct(s, jnp.float32)

_pre = pl.pallas_call(
    _pre_body, out_shape=[_f32(N, H), _f32(N, H)])

_mid = pl.pallas_call(
    _mid_body, out_shape=[_f32(N, H), _f32(N, H)])

_post = pl.pallas_call(
    _post_body, out_shape=_f32(N, H))


def _spmm_body(q_hbm, src_hbm, dst_hbm, zeros_hbm, out0_hbm, out1_hbm,
               idx_sa, idx_da, idx_sb, idx_db, rows_a, rows_b, acc,
               sem_a, sem_b):
    cid = lax.axis_index("c")
    sid = lax.axis_index("s")
    wid = cid * NS + sid
    r0 = sid * RPT
    pltpu.sync_copy(zeros_hbm.at[pl.ds(r0, RPT)], acc.at[pl.ds(r0, RPT)])
    plsc.subcore_barrier()

    base = wid * EPW

    def stage_and_gather(i, idx_s, idx_d, rows, sem):
        off = base + i * CHUNK
        pltpu.sync_copy(src_hbm.at[pl.ds(off, CHUNK)], idx_s)
        pltpu.sync_copy(dst_hbm.at[pl.ds(off, CHUNK)], idx_d)
        pltpu.async_copy(q_hbm.at[idx_s], rows, sem)

    def wait_gather(idx_s, rows, sem):
        pltpu.make_async_copy(q_hbm.at[idx_s], rows, sem).wait()

    def scatter_add(idx_d, rows):
        pltpu.sync_copy(rows, acc.at[idx_d], add=True)

    stage_and_gather(0, idx_sa, idx_da, rows_a, sem_a)

    def body(j, carry):
        i0 = 2 * j
        stage_and_gather(i0 + 1, idx_sb, idx_db, rows_b, sem_b)
        wait_gather(idx_sa, rows_a, sem_a)
        scatter_add(idx_da, rows_a)

        @pl.when(i0 + 2 < NCHUNK)
        def _():
            stage_and_gather(i0 + 2, idx_sa, idx_da, rows_a, sem_a)

        wait_gather(idx_sb, rows_b, sem_b)
        scatter_add(idx_db, rows_b)
        return carry

    lax.fori_loop(0, NCHUNK // 2, body, 0)
    wait_gather(idx_sa, rows_a, sem_a)
    scatter_add(idx_da, rows_a)
    plsc.subcore_barrier()

    @pl.when(cid == 0)
    def _():
        pltpu.sync_copy(acc.at[pl.ds(r0, RPT)], out0_hbm.at[pl.ds(r0, RPT)])

    @pl.when(cid == 1)
    def _():
        pltpu.sync_copy(acc.at[pl.ds(r0, RPT)], out1_hbm.at[pl.ds(r0, RPT)])


_spmm = pl.kernel(
    _spmm_body,
    out_type=[_f32(NPAD, H), _f32(NPAD, H)],
    mesh=plsc.VectorSubcoreMesh(core_axis_name="c", subcore_axis_name="s"),
    scratch_types=[
        pltpu.VMEM((CHUNK,), jnp.int32),
        pltpu.VMEM((CHUNK,), jnp.int32),
        pltpu.VMEM((CHUNK,), jnp.int32),
        pltpu.VMEM((CHUNK,), jnp.int32),
        pltpu.VMEM((CHUNK, H), jnp.float32),
        pltpu.VMEM((CHUNK, H), jnp.float32),
        pltpu.VMEM_SHARED((NPAD, H), jnp.float32),
        pltpu.SemaphoreType.DMA,
        pltpu.SemaphoreType.DMA,
    ],
)


def kernel(x, edge_index, W0_0, b0_0, W0_1, b0_1, W1_0, b1_0, W1_1, b1_1,
           g0, beta0, g1, beta1):
    src = edge_index[0]
    dst = edge_index[1]
    zeros = jnp.zeros((NPAD, H), jnp.float32)

    b0_0r = b0_0.reshape(1, H)
    b0_1r = b0_1.reshape(1, H)
    b1_0r = b1_0.reshape(1, H)
    b1_1r = b1_1.reshape(1, H)
    g0r = g0.reshape(1, H)
    beta0r = beta0.reshape(1, H)
    g1r = g1.reshape(1, H)
    beta1r = beta1.reshape(1, H)

    q0, r1 = _pre(x, W0_0, W1_0[:D])
    p0a, p0b = _spmm(q0, src, dst, zeros)
    c0, q1 = _mid(p0a[:N], p0b[:N], b0_0r, W0_1, b0_1r, g0r, beta0r,
                  r1, W1_0[D:])
    p1a, p1b = _spmm(q1, src, dst, zeros)
    c1 = _post(p1a[:N], p1b[:N], b1_0r, W1_1, b1_1r, g1r, beta1r)
    return jnp.concatenate((x, c0, c1), axis=-1)
